# Initial kernel scaffold; baseline (speedup 1.0000x reference)
import jax
import jax.numpy as jnp
from jax.experimental import pallas as pl


def kernel(idx, embedding):
    raise NotImplementedError("write your pallas kernel here")


# speedup vs baseline: 1.8510x; 1.8510x over previous
import functools

import jax
import jax.numpy as jnp
from jax import lax
from jax.experimental import pallas as pl
from jax.experimental.pallas import tpu as pltpu
from jax.experimental.pallas import tpu_sc as plsc

VOCAB = 100000
D = 64
B = 4096
L = 50
LPAD = 56

NC, NS = 2, 16
NW = NC * NS
BPW = B // NW
LANES = 16


def _sc_lookup_mean(idx_flat, table):
  mesh = plsc.VectorSubcoreMesh(core_axis_name="c", subcore_axis_name="s",
                                num_cores=NC, num_subcores=NS)

  @functools.partial(
      pl.kernel,
      out_type=jax.ShapeDtypeStruct((B, D), jnp.float32),
      mesh=mesh,
      compiler_params=pltpu.CompilerParams(use_tc_tiling_on_sc=False),
      scratch_types=[
          pltpu.VMEM((BPW * LPAD,), jnp.int32),
          pltpu.VMEM((LPAD, D), jnp.float32),
          pltpu.VMEM((BPW, D), jnp.float32),
          pltpu.SemaphoreType.DMA,
      ],
  )
  def k(idx_hbm, table_hbm, out_hbm, idx_v, rows_v, out_v, sem):
    wid = lax.axis_index("s") * NC + lax.axis_index("c")
    base = wid * BPW
    pltpu.sync_copy(idx_hbm.at[pl.ds(base * LPAD, BPW * LPAD)], idx_v)

    def body(e, _):
      pltpu.async_copy(
          table_hbm.at[idx_v.at[pl.ds(e * LPAD, LPAD)]], rows_v, sem).wait()
      for c in range(D // LANES):
        acc = jnp.zeros((LANES,), jnp.float32)
        for r in range(L):
          acc = acc + rows_v[r, pl.ds(c * LANES, LANES)]
        out_v[e, pl.ds(c * LANES, LANES)] = acc * jnp.float32(1.0 / L)
      return 0

    lax.fori_loop(0, BPW, body, 0)
    pltpu.sync_copy(out_v, out_hbm.at[pl.ds(base, BPW)])

  return k(idx_flat, table)


def kernel(idx, embedding):
  idx32 = idx.astype(jnp.int32)
  idx_pad = jnp.pad(idx32, ((0, 0), (0, LPAD - L))).reshape(-1)
  out = _sc_lookup_mean(idx_pad, embedding.astype(jnp.float32))
  return out[:, None, :]
